# Initial kernel scaffold; baseline (speedup 1.0000x reference)
import jax
import jax.numpy as jnp
from jax.experimental import pallas as pl


def kernel(x, edge_index, W1, b1, W2, b2):
    raise NotImplementedError("write your pallas kernel here")


# speedup vs baseline: 4.1761x; 4.1761x over previous
import jax
import jax.numpy as jnp
from jax import lax
from jax.experimental import pallas as pl
from jax.experimental.pallas import tpu as pltpu
from jax.experimental.pallas import tpu_sc as plsc

N = 10000
E = 320000
F = 128
HID = 128
C = 40

NC = 2
NS = 16
NW = NC * NS
CHUNK = 128
EPW = E // NW
KK = -(-EPW // CHUNK)
EPAD = NW * KK * CHUNK
ROWS_PT = 632
NPAD = ROWS_PT * NS


def _make_scatter_add(D):
    mesh = plsc.VectorSubcoreMesh(core_axis_name="c", subcore_axis_name="s")

    def body(table, src_idx, dst_idx, zeros_hbm, out, src_v, dst_v, rows_v,
             acc, sem):
        cid = lax.axis_index("c")
        sid = lax.axis_index("s")
        r0 = sid * ROWS_PT
        pltpu.sync_copy(zeros_hbm.at[pl.ds(r0, ROWS_PT)],
                        acc.at[pl.ds(r0, ROWS_PT)])
        pltpu.sync_copy(src_idx.at[cid, sid], src_v)
        pltpu.sync_copy(dst_idx.at[cid, sid], dst_v)
        plsc.subcore_barrier()

        @pl.loop(0, KK)
        def _chunk(j):
            pltpu.async_copy(table.at[src_v.at[j]], rows_v, sem).wait()
            pltpu.sync_copy(rows_v, acc.at[dst_v.at[j]], add=True)

        plsc.subcore_barrier()
        pltpu.sync_copy(acc.at[pl.ds(r0, ROWS_PT)],
                        out.at[cid, pl.ds(r0, ROWS_PT)])

    return pl.kernel(
        body,
        out_type=jax.ShapeDtypeStruct((NC, NPAD, D), jnp.float32),
        mesh=mesh,
        scratch_types=[
            pltpu.VMEM((KK, CHUNK), jnp.int32),
            pltpu.VMEM((KK, CHUNK), jnp.int32),
            pltpu.VMEM((CHUNK, D), jnp.float32),
            pltpu.VMEM_SHARED((NPAD, D), jnp.float32),
            pltpu.SemaphoreType.DMA,
        ],
    )


_scatter = _make_scatter_add(F)

BM = 1000


def _mm1_body(x_ref, p_ref, w1_ref, b1_ref, h_ref):
    s = x_ref[...] + p_ref[0] + p_ref[1]
    h = jnp.dot(s, w1_ref[...], preferred_element_type=jnp.float32)
    h_ref[...] = jnp.maximum(h + b1_ref[...], 0.0)


_mm1 = pl.pallas_call(
    _mm1_body,
    grid=(N // BM,),
    in_specs=[
        pl.BlockSpec((BM, F), lambda i: (i, 0)),
        pl.BlockSpec((NC, BM, F), lambda i: (0, i, 0)),
        pl.BlockSpec((F, HID), lambda i: (0, 0)),
        pl.BlockSpec((1, HID), lambda i: (0, 0)),
    ],
    out_specs=pl.BlockSpec((BM, HID), lambda i: (i, 0)),
    out_shape=jax.ShapeDtypeStruct((N, HID), jnp.float32),
)


def _mm2_body(h_ref, q_ref, w2_ref, b2_ref, o_ref):
    s = h_ref[...] + q_ref[0] + q_ref[1]
    o = jnp.dot(s, w2_ref[...], preferred_element_type=jnp.float32)
    o_ref[...] = o + b2_ref[...]


_mm2 = pl.pallas_call(
    _mm2_body,
    grid=(N // BM,),
    in_specs=[
        pl.BlockSpec((BM, HID), lambda i: (i, 0)),
        pl.BlockSpec((NC, BM, HID), lambda i: (0, i, 0)),
        pl.BlockSpec((HID, C), lambda i: (0, 0)),
        pl.BlockSpec((1, C), lambda i: (0, 0)),
    ],
    out_specs=pl.BlockSpec((BM, C), lambda i: (i, 0)),
    out_shape=jax.ShapeDtypeStruct((N, C), jnp.float32),
)


def kernel(x, edge_index, W1, b1, W2, b2):
    src = edge_index[0].astype(jnp.int32)
    dst = edge_index[1].astype(jnp.int32)
    pad = EPAD - E
    src_p = jnp.concatenate(
        [src, jnp.zeros((pad,), jnp.int32)]).reshape(NC, NS, KK, CHUNK)
    dst_p = jnp.concatenate(
        [dst, jnp.full((pad,), N, jnp.int32)]).reshape(NC, NS, KK, CHUNK)
    zeros = jnp.zeros((NPAD, F), jnp.float32)

    p = _scatter(x, src_p, dst_p, zeros)
    h = _mm1(x, p, W1, b1.reshape(1, HID))
    q = _scatter(h, src_p, dst_p, zeros)
    return _mm2(h, q, W2, b2.reshape(1, C))
